# baseline (speedup 1.0000x reference)
import jax
import jax.numpy as jnp
from jax import lax
from jax.experimental import pallas as pl
from jax.experimental.pallas import tpu as pltpu

N_INPUTS = 4096
N_NEURONS = 4096
N_OUTPUTS = 1024
BATCH = 4096

WC = 128
TK = 256
TB = 512
N_WC = N_OUTPUTS // WC
N_K = N_INPUTS // TK
N_B = BATCH // TB


def _body(wout_hbm, w_hbm, x_hbm, out_hbm,
          wstage, wout_bf, wbuf, xbuf, obuf, t,
          sem_wout, sem_w, sem_x, sem_o):

    def wout_dma(c, s):
        return pltpu.make_async_copy(
            wout_hbm.at[pl.ds(c * WC, WC), :], wstage.at[s], sem_wout.at[s])

    def w_dma(i, s):
        return pltpu.make_async_copy(
            w_hbm.at[:, pl.ds(i * TK, TK)], wbuf.at[s], sem_w.at[s])

    def x_dma(j, s):
        return pltpu.make_async_copy(
            x_hbm.at[pl.ds(j * TB, TB), :], xbuf.at[s], sem_x.at[s])

    def o_dma(j, s):
        return pltpu.make_async_copy(
            obuf.at[s], out_hbm.at[:, pl.ds(j * TB, TB)], sem_o.at[s])

    wout_dma(0, 0).start()
    wout_dma(1, 1).start()
    w_dma(0, 0).start()
    w_dma(1, 1).start()
    x_dma(0, 0).start()
    x_dma(1, 1).start()

    for c in range(N_WC):
        s = c % 2
        wout_dma(c, s).wait()
        wout_bf[pl.ds(c * WC, WC), :] = wstage[s].astype(jnp.bfloat16)
        if c + 2 < N_WC:
            wout_dma(c + 2, s).start()

    for i in range(N_K):
        s = i % 2
        w_dma(i, s).wait()
        acc = jnp.dot(wout_bf[...], wbuf[s].astype(jnp.bfloat16),
                      preferred_element_type=jnp.float32)
        t[:, pl.ds(i * TK, TK)] = acc.astype(jnp.bfloat16)
        if i + 2 < N_K:
            w_dma(i + 2, s).start()

    for j in range(N_B):
        s = j % 2
        x_dma(j, s).wait()
        if j >= 2:
            o_dma(j - 2, s).wait()
        obuf[s] = lax.dot_general(
            t[...], xbuf[s].astype(jnp.bfloat16),
            dimension_numbers=(((1,), (1,)), ((), ())),
            preferred_element_type=jnp.float32)
        o_dma(j, s).start()
        if j + 2 < N_B:
            x_dma(j + 2, s).start()

    o_dma(N_B - 2, (N_B - 2) % 2).wait()
    o_dma(N_B - 1, (N_B - 1) % 2).wait()


@jax.jit
def kernel(x, w, wout):
    out = pl.pallas_call(
        _body,
        in_specs=[
            pl.BlockSpec(memory_space=pl.ANY),
            pl.BlockSpec(memory_space=pl.ANY),
            pl.BlockSpec(memory_space=pl.ANY),
        ],
        out_specs=pl.BlockSpec(memory_space=pl.ANY),
        out_shape=jax.ShapeDtypeStruct((N_OUTPUTS, BATCH), jnp.float32),
        scratch_shapes=[
            pltpu.VMEM((2, WC, N_NEURONS), jnp.float32),
            pltpu.VMEM((N_OUTPUTS, N_NEURONS), jnp.bfloat16),
            pltpu.VMEM((2, N_NEURONS, TK), jnp.float32),
            pltpu.VMEM((2, TB, N_INPUTS), jnp.float32),
            pltpu.VMEM((2, N_OUTPUTS, TB), jnp.float32),
            pltpu.VMEM((N_OUTPUTS, N_INPUTS), jnp.bfloat16),
            pltpu.SemaphoreType.DMA((2,)),
            pltpu.SemaphoreType.DMA((2,)),
            pltpu.SemaphoreType.DMA((2,)),
            pltpu.SemaphoreType.DMA((2,)),
        ],
    )(wout, w, x)
    return out


# speedup vs baseline: 1.3135x; 1.3135x over previous
import jax
import jax.numpy as jnp
from jax import lax
from jax.experimental import pallas as pl
from jax.experimental.pallas import tpu as pltpu

N_INPUTS = 4096
N_NEURONS = 4096
N_OUTPUTS = 1024
BATCH = 4096

TK1 = 512
TB = 512
K_TILES = N_INPUTS // TK1
B_TILES = BATCH // TB


def _body(wout_ref, w_ref, x_ref, out_ref, t_ref):
    i = pl.program_id(0)

    @pl.when(i < K_TILES)
    def _build_t():
        acc = jnp.dot(wout_ref[...], w_ref[...].astype(jnp.bfloat16),
                      preferred_element_type=jnp.float32)
        t_ref[:, pl.ds(i * TK1, TK1)] = acc.astype(jnp.bfloat16)

    @pl.when(i >= K_TILES)
    def _emit_out():
        out_ref[...] = lax.dot_general(
            t_ref[...], x_ref[...].astype(jnp.bfloat16),
            dimension_numbers=(((1,), (1,)), ((), ())),
            preferred_element_type=jnp.float32)


@jax.jit
def kernel(x, w, wout):
    kmax = K_TILES - 1
    wout_bf = wout.astype(jnp.bfloat16)
    out = pl.pallas_call(
        _body,
        grid=(K_TILES + B_TILES,),
        in_specs=[
            pl.BlockSpec((N_OUTPUTS, N_NEURONS), lambda i: (0, 0)),
            pl.BlockSpec((N_NEURONS, TK1),
                         lambda i: (0, jnp.minimum(i, kmax))),
            pl.BlockSpec((TB, N_INPUTS),
                         lambda i: (jnp.maximum(i - K_TILES, 0), 0)),
        ],
        out_specs=pl.BlockSpec((N_OUTPUTS, TB),
                               lambda i: (0, jnp.maximum(i - K_TILES, 0))),
        out_shape=jax.ShapeDtypeStruct((N_OUTPUTS, BATCH), jnp.float32),
        scratch_shapes=[pltpu.VMEM((N_OUTPUTS, N_INPUTS), jnp.bfloat16)],
    )(wout_bf, w, x)
    return out
